# baseline (speedup 1.0000x reference)
import functools

import jax
import jax.numpy as jnp
from jax import lax
from jax.experimental import pallas as pl
from jax.experimental.pallas import tpu as pltpu
from jax.experimental.pallas import tpu_sc as plsc

_VOCAB = 1000000
_EMBED = 300
_BATCH = 1024
_SEQ = 512
_HIDDEN = 16

_NC, _NS = 2, 16
_NW = _NC * _NS
_RPW = _BATCH // _NW
_CHUNK = 128
_NCH = _SEQ // _CHUNK
_NFULL = _EMBED // 16
_TAIL_OFF = _EMBED - 16


def _bag_body(tok_hbm, table_hbm, out_hbm, tok_v, rows_v, out_v, sem):
    wid = lax.axis_index("s") * _NC + lax.axis_index("c")
    tpw = _RPW * _SEQ
    pltpu.sync_copy(tok_hbm.at[pl.ds(wid * tpw, tpw)], tok_v)

    def row_body(r, carry):
        def chunk_body(c, acc):
            idx = tok_v.at[pl.ds(r * _SEQ + c * _CHUNK, _CHUNK)]
            pltpu.async_copy(table_hbm.at[idx], rows_v, sem).wait()

            def tok_body(t, a):
                full = tuple(
                    a[i] + rows_v[t, pl.ds(i * 16, 16)] for i in range(_NFULL)
                )
                tail = a[_NFULL] + rows_v[t, pl.ds(_TAIL_OFF, 16)]
                return full + (tail,)

            return lax.fori_loop(0, _CHUNK, tok_body, acc)

        zero = jnp.zeros((16,), jnp.float32)
        acc = lax.fori_loop(0, _NCH, chunk_body, (zero,) * (_NFULL + 1))
        obase = r * _EMBED
        out_v[pl.ds(obase + _TAIL_OFF, 16)] = acc[_NFULL]
        for i in range(_NFULL):
            out_v[pl.ds(obase + i * 16, 16)] = acc[i]
        return carry

    lax.fori_loop(0, _RPW, row_body, 0)
    opw = _RPW * _EMBED
    pltpu.sync_copy(out_v, out_hbm.at[pl.ds(wid * opw, opw)])


_bag = functools.partial(
    pl.kernel,
    out_type=jax.ShapeDtypeStruct((_BATCH * _EMBED,), jnp.float32),
    mesh=plsc.VectorSubcoreMesh(core_axis_name="c", subcore_axis_name="s"),
    scratch_types=[
        pltpu.VMEM((_RPW * _SEQ,), jnp.int32),
        pltpu.VMEM((_CHUNK, _EMBED), jnp.float32),
        pltpu.VMEM((_RPW * _EMBED,), jnp.float32),
        pltpu.SemaphoreType.DMA,
    ],
    compiler_params=pltpu.CompilerParams(use_tc_tiling_on_sc=False),
)(_bag_body)


def _mlp_body(x_ref, w1_ref, b1_ref, w2_ref, b2_ref, o_ref):
    x = x_ref[:] * jnp.float32(1.0 / _SEQ)
    h = jnp.dot(x, w1_ref[:], preferred_element_type=jnp.float32) + b1_ref[:]
    h = jnp.maximum(h, 0.0)
    z = jnp.dot(h, w2_ref[:], preferred_element_type=jnp.float32) + b2_ref[:]
    o_ref[:] = 1.0 / (1.0 + jnp.exp(-z))


def kernel(tokens, emb_table, W1, b1, W2, b2):
    pooled_sum = _bag(tokens.reshape(-1), emb_table)
    out = pl.pallas_call(
        _mlp_body,
        out_shape=jax.ShapeDtypeStruct((_BATCH, 1), jnp.float32),
    )(pooled_sum.reshape(_BATCH, _EMBED), W1, b1.reshape(1, _HIDDEN),
      W2, b2.reshape(1, 1))
    return out


# speedup vs baseline: 3.1559x; 3.1559x over previous
import functools

import jax
import jax.numpy as jnp
from jax import lax
from jax.experimental import pallas as pl
from jax.experimental.pallas import tpu as pltpu
from jax.experimental.pallas import tpu_sc as plsc

_VOCAB = 1000000
_EMBED = 300
_BATCH = 1024
_SEQ = 512
_HIDDEN = 16

_NC, _NS = 2, 16
_NW = _NC * _NS
_RPW = _BATCH // _NW
_CHUNK = 128
_NCH = _SEQ // _CHUNK
_NG = _RPW * _NCH
_PROJ_BLK = 2000


def _proj_body(x_ref, w1_ref, o_ref):
    w1s = (w1_ref[:] * jnp.float32(1.0 / _SEQ)).astype(jnp.bfloat16)
    h = jnp.dot(x_ref[:].astype(jnp.bfloat16), w1s,
                preferred_element_type=jnp.float32)
    o_ref[:] = jnp.concatenate([h] * 8, axis=1)


_proj = pl.pallas_call(
    _proj_body,
    grid=(_VOCAB // _PROJ_BLK,),
    in_specs=[
        pl.BlockSpec((_PROJ_BLK, _EMBED), lambda i: (i, 0)),
        pl.BlockSpec((_EMBED, _HIDDEN), lambda i: (0, 0)),
    ],
    out_specs=pl.BlockSpec((_PROJ_BLK, 128), lambda i: (i, 0)),
    out_shape=jax.ShapeDtypeStruct((_VOCAB, 128), jnp.float32),
)


def _bag_body(tok_hbm, p_hbm, out_hbm, tok_v, rows_v, out_v, sem0, sem1):
    wid = lax.axis_index("s") * _NC + lax.axis_index("c")
    tpw = _RPW * _SEQ
    pltpu.sync_copy(tok_hbm.at[pl.ds(wid * tpw, tpw)], tok_v)

    sems = (sem0, sem1)

    def gather(g, slot):
        idx = tok_v.at[pl.ds(g * _CHUNK, _CHUNK)]
        return pltpu.async_copy(p_hbm.at[idx], rows_v.at[slot], sems[slot])

    def gather_wait(g, slot):
        idx = tok_v.at[pl.ds(g * _CHUNK, _CHUNK)]
        pltpu.make_async_copy(p_hbm.at[idx], rows_v.at[slot], sems[slot]).wait()

    gather(0, 0)
    gather(1, 1)

    def row_body(r, _):
        def acc_chunk(slot, acc):
            def tok_body(t, a):
                return a + rows_v[slot, t, pl.ds(0, 16)]
            return lax.fori_loop(0, _CHUNK, tok_body, acc)

        acc = jnp.zeros((16,), jnp.float32)
        for c in range(_NCH):
            g = r * _NCH + c
            slot = c % 2
            gather_wait(g, slot)

            @pl.when(g + 2 < _NG)
            def _():
                gather(g + 2, slot)

            acc = acc_chunk(slot, acc)
        out_v[pl.ds(r * _HIDDEN, _HIDDEN)] = acc
        return 0

    lax.fori_loop(0, _RPW, row_body, 0)
    opw = _RPW * _HIDDEN
    pltpu.sync_copy(out_v, out_hbm.at[pl.ds(wid * opw, opw)])


_bag = functools.partial(
    pl.kernel,
    out_type=jax.ShapeDtypeStruct((_BATCH * _HIDDEN,), jnp.float32),
    mesh=plsc.VectorSubcoreMesh(core_axis_name="c", subcore_axis_name="s"),
    scratch_types=[
        pltpu.VMEM((_RPW * _SEQ,), jnp.int32),
        pltpu.VMEM((2, _CHUNK, 128), jnp.float32),
        pltpu.VMEM((_RPW * _HIDDEN,), jnp.float32),
        pltpu.SemaphoreType.DMA,
        pltpu.SemaphoreType.DMA,
    ],
)(_bag_body)


def _head_body(h_ref, b1_ref, w2_ref, b2_ref, o_ref):
    h = jnp.maximum(h_ref[:] + b1_ref[:], 0.0)
    z = jnp.dot(h, w2_ref[:], preferred_element_type=jnp.float32) + b2_ref[:]
    o_ref[:] = 1.0 / (1.0 + jnp.exp(-z))


def kernel(tokens, emb_table, W1, b1, W2, b2):
    p4 = _proj(emb_table, W1)
    h_sum = _bag(tokens.reshape(-1), p4)
    out = pl.pallas_call(
        _head_body,
        out_shape=jax.ShapeDtypeStruct((_BATCH, 1), jnp.float32),
    )(h_sum.reshape(_BATCH, _HIDDEN), b1.reshape(1, _HIDDEN),
      W2, b2.reshape(1, 1))
    return out


# speedup vs baseline: 3.6513x; 1.1570x over previous
import functools

import jax
import jax.numpy as jnp
from jax import lax
from jax.experimental import pallas as pl
from jax.experimental.pallas import tpu as pltpu
from jax.experimental.pallas import tpu_sc as plsc

_VOCAB = 1000000
_EMBED = 300
_BATCH = 1024
_SEQ = 512
_HIDDEN = 16

_NC, _NS = 2, 16
_NW = _NC * _NS
_RPW = _BATCH // _NW
_CHUNK = 128
_NCH = _SEQ // _CHUNK
_NG = _RPW * _NCH
_PROJ_BLK = 2000


def _proj_body(x_ref, w1_ref, o_ref):
    o_ref[:] = jnp.dot(x_ref[:].astype(jnp.bfloat16), w1_ref[:],
                       preferred_element_type=jnp.float32)


_proj = pl.pallas_call(
    _proj_body,
    grid=(_VOCAB // _PROJ_BLK,),
    in_specs=[
        pl.BlockSpec((_PROJ_BLK, _EMBED), lambda i: (i, 0)),
        pl.BlockSpec((_EMBED, 128), lambda i: (0, 0)),
    ],
    out_specs=pl.BlockSpec((_PROJ_BLK, 128), lambda i: (i, 0)),
    out_shape=jax.ShapeDtypeStruct((_VOCAB, 128), jnp.float32),
)


def _bag_body(tok_hbm, p_hbm, out_hbm, tok_v, rows_v, out_v, sem0, sem1):
    wid = lax.axis_index("s") * _NC + lax.axis_index("c")
    tpw = _RPW * _SEQ
    pltpu.sync_copy(tok_hbm.at[pl.ds(wid * tpw, tpw)], tok_v)

    sems = (sem0, sem1)

    def gather(g, slot):
        idx = tok_v.at[pl.ds(g * _CHUNK, _CHUNK)]
        return pltpu.async_copy(p_hbm.at[idx], rows_v.at[slot], sems[slot])

    def gather_wait(g, slot):
        idx = tok_v.at[pl.ds(g * _CHUNK, _CHUNK)]
        pltpu.make_async_copy(p_hbm.at[idx], rows_v.at[slot], sems[slot]).wait()

    gather(0, 0)
    gather(1, 1)

    def row_body(r, _):
        def acc_chunk(slot, acc):
            def tok_body(t, a):
                return a + rows_v[slot, t, pl.ds(0, 16)]
            return lax.fori_loop(0, _CHUNK, tok_body, acc)

        acc = jnp.zeros((16,), jnp.float32)
        for c in range(_NCH):
            g = r * _NCH + c
            slot = c % 2
            gather_wait(g, slot)

            @pl.when(g + 2 < _NG)
            def _():
                gather(g + 2, slot)

            acc = acc_chunk(slot, acc)
        out_v[pl.ds(r * _HIDDEN, _HIDDEN)] = acc
        return 0

    lax.fori_loop(0, _RPW, row_body, 0)
    opw = _RPW * _HIDDEN
    pltpu.sync_copy(out_v, out_hbm.at[pl.ds(wid * opw, opw)])


_bag = functools.partial(
    pl.kernel,
    out_type=jax.ShapeDtypeStruct((_BATCH * _HIDDEN,), jnp.float32),
    mesh=plsc.VectorSubcoreMesh(core_axis_name="c", subcore_axis_name="s"),
    scratch_types=[
        pltpu.VMEM((_RPW * _SEQ,), jnp.int32),
        pltpu.VMEM((2, _CHUNK, 128), jnp.float32),
        pltpu.VMEM((_RPW * _HIDDEN,), jnp.float32),
        pltpu.SemaphoreType.DMA,
        pltpu.SemaphoreType.DMA,
    ],
)(_bag_body)


def _head_body(h_ref, b1_ref, w2_ref, b2_ref, o_ref):
    h = jnp.maximum(h_ref[:] + b1_ref[:], 0.0)
    z = jnp.dot(h, w2_ref[:], preferred_element_type=jnp.float32) + b2_ref[:]
    o_ref[:] = 1.0 / (1.0 + jnp.exp(-z))


def kernel(tokens, emb_table, W1, b1, W2, b2):
    w1rep = jnp.tile(W1 * jnp.float32(1.0 / _SEQ), (1, 8)).astype(jnp.bfloat16)
    p4 = _proj(emb_table, w1rep)
    h_sum = _bag(tokens.reshape(-1), p4)
    out = pl.pallas_call(
        _head_body,
        out_shape=jax.ShapeDtypeStruct((_BATCH, 1), jnp.float32),
    )(h_sum.reshape(_BATCH, _HIDDEN), b1.reshape(1, _HIDDEN),
      W2, b2.reshape(1, 1))
    return out


# speedup vs baseline: 3.9943x; 1.0939x over previous
import functools

import jax
import jax.numpy as jnp
from jax import lax
from jax.experimental import pallas as pl
from jax.experimental.pallas import tpu as pltpu
from jax.experimental.pallas import tpu_sc as plsc

_VOCAB = 1000000
_EMBED = 300
_BATCH = 1024
_SEQ = 512
_HIDDEN = 16

_NC, _NS = 2, 16
_NW = _NC * _NS
_RPW = _BATCH // _NW
_CHUNK = 128
_NCH = _SEQ // _CHUNK
_NG = _RPW * _NCH
_PROJ_BLK = 8000


def _proj_body(x_ref, w1_ref, o_ref):
    o_ref[:] = jnp.dot(x_ref[:].astype(jnp.bfloat16), w1_ref[:],
                       preferred_element_type=jnp.float32)


_proj = pl.pallas_call(
    _proj_body,
    grid=(_VOCAB // _PROJ_BLK,),
    in_specs=[
        pl.BlockSpec((_PROJ_BLK, _EMBED), lambda i: (i, 0)),
        pl.BlockSpec((_EMBED, 128), lambda i: (0, 0)),
    ],
    out_specs=pl.BlockSpec((_PROJ_BLK, 128), lambda i: (i, 0)),
    out_shape=jax.ShapeDtypeStruct((_VOCAB, 128), jnp.float32),
)


def _bag_body(tok_hbm, p_hbm, out_hbm, tok_v, rows_v, out_v, sem0, sem1):
    wid = lax.axis_index("s") * _NC + lax.axis_index("c")
    tpw = _RPW * _SEQ
    pltpu.sync_copy(tok_hbm.at[pl.ds(wid * tpw, tpw)], tok_v)

    sems = (sem0, sem1)

    def gather(g, slot):
        idx = tok_v.at[pl.ds(g * _CHUNK, _CHUNK)]
        return pltpu.async_copy(p_hbm.at[idx], rows_v.at[slot], sems[slot])

    def gather_wait(g, slot):
        idx = tok_v.at[pl.ds(g * _CHUNK, _CHUNK)]
        pltpu.make_async_copy(p_hbm.at[idx], rows_v.at[slot], sems[slot]).wait()

    gather(0, 0)
    gather(1, 1)

    def row_body(r, _):
        def acc_chunk(slot, acc):
            def tok_body(t, a):
                return a + rows_v[slot, t, pl.ds(0, 16)]
            return lax.fori_loop(0, _CHUNK, tok_body, acc)

        acc = jnp.zeros((16,), jnp.float32)
        for c in range(_NCH):
            g = r * _NCH + c
            slot = c % 2
            gather_wait(g, slot)

            @pl.when(g + 2 < _NG)
            def _():
                gather(g + 2, slot)

            acc = acc_chunk(slot, acc)
        out_v[pl.ds(r * _HIDDEN, _HIDDEN)] = acc
        return 0

    lax.fori_loop(0, _RPW, row_body, 0)
    opw = _RPW * _HIDDEN
    pltpu.sync_copy(out_v, out_hbm.at[pl.ds(wid * opw, opw)])


_bag = functools.partial(
    pl.kernel,
    out_type=jax.ShapeDtypeStruct((_BATCH * _HIDDEN,), jnp.float32),
    mesh=plsc.VectorSubcoreMesh(core_axis_name="c", subcore_axis_name="s"),
    scratch_types=[
        pltpu.VMEM((_RPW * _SEQ,), jnp.int32),
        pltpu.VMEM((2, _CHUNK, 128), jnp.float32),
        pltpu.VMEM((_RPW * _HIDDEN,), jnp.float32),
        pltpu.SemaphoreType.DMA,
        pltpu.SemaphoreType.DMA,
    ],
)(_bag_body)


def _head_body(h_ref, b1_ref, w2_ref, b2_ref, o_ref):
    h = jnp.maximum(h_ref[:] + b1_ref[:], 0.0)
    z = jnp.dot(h, w2_ref[:], preferred_element_type=jnp.float32) + b2_ref[:]
    o_ref[:] = 1.0 / (1.0 + jnp.exp(-z))


def kernel(tokens, emb_table, W1, b1, W2, b2):
    w1rep = jnp.tile(W1 * jnp.float32(1.0 / _SEQ), (1, 8)).astype(jnp.bfloat16)
    p4 = _proj(emb_table, w1rep)
    h_sum = _bag(tokens.reshape(-1), p4)
    out = pl.pallas_call(
        _head_body,
        out_shape=jax.ShapeDtypeStruct((_BATCH, 1), jnp.float32),
    )(h_sum.reshape(_BATCH, _HIDDEN), b1.reshape(1, _HIDDEN),
      W2, b2.reshape(1, 1))
    return out


# speedup vs baseline: 4.2383x; 1.0611x over previous
import functools

import jax
import jax.numpy as jnp
from jax import lax
from jax.experimental import pallas as pl
from jax.experimental.pallas import tpu as pltpu
from jax.experimental.pallas import tpu_sc as plsc

_VOCAB = 1000000
_EMBED = 300
_BATCH = 1024
_SEQ = 512
_HIDDEN = 16

_NC, _NS = 2, 16
_NW = _NC * _NS
_RPW = _BATCH // _NW
_CHUNK = 128
_NCH = _SEQ // _CHUNK
_NG = _RPW * _NCH
_BAND = 1000
_NBAND = 128 // _HIDDEN
_PBLK = _BAND * _NBAND
_P2ROWS = _VOCAB // _NBAND


def _proj_body(*refs):
    xs, w_ref, o_ref = refs[:_NBAND], refs[_NBAND], refs[_NBAND + 1]
    acc = jnp.zeros((_BAND, 128), jnp.float32)
    for j in range(_NBAND):
        wj = w_ref[pl.ds(j * _EMBED, _EMBED), :]
        acc = acc + jnp.dot(xs[j][:].astype(jnp.bfloat16), wj,
                            preferred_element_type=jnp.float32)
    o_ref[:] = acc


_proj = pl.pallas_call(
    _proj_body,
    grid=(_VOCAB // _PBLK,),
    in_specs=[
        pl.BlockSpec((_BAND, _EMBED), lambda i, j=j: (_NBAND * i + j, 0))
        for j in range(_NBAND)
    ] + [pl.BlockSpec((_NBAND * _EMBED, 128), lambda i: (0, 0))],
    out_specs=pl.BlockSpec((_BAND, 128), lambda i: (i, 0)),
    out_shape=jax.ShapeDtypeStruct((_P2ROWS, 128), jnp.float32),
)


def _tokprep_body(t_ref, row_ref, colb_ref):
    t = t_ref[:]
    blk = t // _PBLK
    band = (t // _BAND) % _NBAND
    r = t % _BAND
    row_ref[:] = blk * _BAND + r
    colb_ref[:] = band * _HIDDEN


_tokprep = pl.pallas_call(
    _tokprep_body,
    out_shape=(
        jax.ShapeDtypeStruct((_BATCH * _SEQ // 128, 128), jnp.int32),
        jax.ShapeDtypeStruct((_BATCH * _SEQ // 128, 128), jnp.int32),
    ),
)


def _bag_body(row_hbm, colb_hbm, p2_hbm, out_hbm,
              row_v, colb_v, rows0_v, rows1_v, tr_v, out_v, sem0, sem1):
    wid = lax.axis_index("s") * _NC + lax.axis_index("c")
    tpw = _RPW * _SEQ
    pltpu.sync_copy(row_hbm.at[pl.ds(wid * tpw, tpw)], row_v)
    pltpu.sync_copy(colb_hbm.at[pl.ds(wid * tpw, tpw)], colb_v)

    sems = (sem0, sem1)
    bufs = (rows0_v, rows1_v)

    def gather(g, slot):
        idx = row_v.at[pl.ds(g * _CHUNK, _CHUNK)]
        return pltpu.async_copy(p2_hbm.at[idx], bufs[slot], sems[slot])

    def gather_wait(g, slot):
        idx = row_v.at[pl.ds(g * _CHUNK, _CHUNK)]
        pltpu.make_async_copy(p2_hbm.at[idx], bufs[slot], sems[slot]).wait()

    gather(0, 0)
    gather(1, 1)

    lane = lax.iota(jnp.int32, 16)

    def row_body(r, _):
        accT = (jnp.zeros((16,), jnp.float32),) * 16
        for c in range(_NCH):
            g = r * _NCH + c
            slot = c % 2
            gather_wait(g, slot)

            @pl.when(g + 2 < _NG)
            def _():
                gather(g + 2, slot)

            buf = bufs[slot]

            def group_body(gi, a):
                off = g * _CHUNK + gi * 16
                colb = colb_v[pl.ds(off, 16)]
                rowi = lane + gi * 16
                return tuple(
                    a[l] + plsc.load_gather(buf, [rowi, colb + l])
                    for l in range(16)
                )

            accT = lax.fori_loop(0, _CHUNK // 16, group_body, accT)

        for l in range(16):
            plsc.store_scatter(tr_v, [lane, jnp.full((16,), l, jnp.int32)],
                               accT[l])
        out = tr_v[0, :]
        for k in range(1, 16):
            out = out + tr_v[k, :]
        out_v[pl.ds(r * _HIDDEN, _HIDDEN)] = out
        return 0

    lax.fori_loop(0, _RPW, row_body, 0)
    opw = _RPW * _HIDDEN
    pltpu.sync_copy(out_v, out_hbm.at[pl.ds(wid * opw, opw)])


_bag = functools.partial(
    pl.kernel,
    out_type=jax.ShapeDtypeStruct((_BATCH * _HIDDEN,), jnp.float32),
    mesh=plsc.VectorSubcoreMesh(core_axis_name="c", subcore_axis_name="s"),
    scratch_types=[
        pltpu.VMEM((_RPW * _SEQ,), jnp.int32),
        pltpu.VMEM((_RPW * _SEQ,), jnp.int32),
        pltpu.VMEM((_CHUNK, 128), jnp.float32),
        pltpu.VMEM((_CHUNK, 128), jnp.float32),
        pltpu.VMEM((16, 16), jnp.float32),
        pltpu.VMEM((_RPW * _HIDDEN,), jnp.float32),
        pltpu.SemaphoreType.DMA,
        pltpu.SemaphoreType.DMA,
    ],
    compiler_params=pltpu.CompilerParams(needs_layout_passes=False),
)(_bag_body)


def _head_body(h_ref, b1_ref, w2_ref, b2_ref, o_ref):
    h = jnp.maximum(h_ref[:] + b1_ref[:], 0.0)
    z = jnp.dot(h, w2_ref[:], preferred_element_type=jnp.float32) + b2_ref[:]
    o_ref[:] = 1.0 / (1.0 + jnp.exp(-z))


def kernel(tokens, emb_table, W1, b1, W2, b2):
    w1s = W1 * jnp.float32(1.0 / _SEQ)
    w1bd = jnp.zeros((_NBAND * _EMBED, 128), jnp.float32)
    for j in range(_NBAND):
        w1bd = w1bd.at[j * _EMBED:(j + 1) * _EMBED,
                       j * _HIDDEN:(j + 1) * _HIDDEN].set(w1s)
    w1bd = w1bd.astype(jnp.bfloat16)

    p2 = _proj(*([emb_table] * _NBAND), w1bd)
    grow, gcolb = _tokprep(tokens.reshape(-1, 128))
    h_sum = _bag(grow.reshape(-1), gcolb.reshape(-1), p2)
    out = pl.pallas_call(
        _head_body,
        out_shape=jax.ShapeDtypeStruct((_BATCH, 1), jnp.float32),
    )(h_sum.reshape(_BATCH, _HIDDEN), b1.reshape(1, _HIDDEN),
      W2, b2.reshape(1, 1))
    return out


# speedup vs baseline: 4.2416x; 1.0008x over previous
import functools

import jax
import jax.numpy as jnp
from jax import lax
from jax.experimental import pallas as pl
from jax.experimental.pallas import tpu as pltpu
from jax.experimental.pallas import tpu_sc as plsc

_VOCAB = 1000000
_EMBED = 300
_BATCH = 1024
_SEQ = 512
_HIDDEN = 16

_NC, _NS = 2, 16
_NW = _NC * _NS
_RPW = _BATCH // _NW
_CHUNK = 128
_NCH = _SEQ // _CHUNK
_NG = _RPW * _NCH
_BAND = 1000
_NBAND = 128 // _HIDDEN
_PBLK = _BAND * _NBAND
_P2ROWS = _VOCAB // _NBAND


_TROWS = _BATCH * _SEQ // 128


def _proj_body(*refs):
    xs, w_ref, t_ref = refs[:_NBAND], refs[_NBAND], refs[_NBAND + 1]
    o_ref, row_ref, colb_ref = refs[_NBAND + 2:]
    acc = jnp.zeros((_BAND, 128), jnp.float32)
    for j in range(_NBAND):
        wj = w_ref[pl.ds(j * _EMBED, _EMBED), :]
        acc = acc + jnp.dot(xs[j][:].astype(jnp.bfloat16), wj,
                            preferred_element_type=jnp.float32)
    o_ref[:] = acc

    @pl.when(pl.program_id(0) == 0)
    def _():
        t = t_ref[:]
        blk = t // _PBLK
        band = (t // _BAND) % _NBAND
        r = t % _BAND
        row_ref[:] = blk * _BAND + r
        colb_ref[:] = band * _HIDDEN


_proj = pl.pallas_call(
    _proj_body,
    grid=(_VOCAB // _PBLK,),
    in_specs=[
        pl.BlockSpec((_BAND, _EMBED), lambda i, j=j: (_NBAND * i + j, 0))
        for j in range(_NBAND)
    ] + [
        pl.BlockSpec((_NBAND * _EMBED, 128), lambda i: (0, 0)),
        pl.BlockSpec((_TROWS, 128), lambda i: (0, 0)),
    ],
    out_specs=[
        pl.BlockSpec((_BAND, 128), lambda i: (i, 0)),
        pl.BlockSpec((_TROWS, 128), lambda i: (0, 0)),
        pl.BlockSpec((_TROWS, 128), lambda i: (0, 0)),
    ],
    out_shape=[
        jax.ShapeDtypeStruct((_P2ROWS, 128), jnp.float32),
        jax.ShapeDtypeStruct((_TROWS, 128), jnp.int32),
        jax.ShapeDtypeStruct((_TROWS, 128), jnp.int32),
    ],
)


def _bag_body(row_hbm, colb_hbm, p2_hbm, out_hbm,
              row_v, colb_v, rows0_v, rows1_v, tr_v, out_v, sem0, sem1):
    wid = lax.axis_index("s") * _NC + lax.axis_index("c")
    tpw = _RPW * _SEQ
    pltpu.sync_copy(row_hbm.at[pl.ds(wid * tpw, tpw)], row_v)
    pltpu.sync_copy(colb_hbm.at[pl.ds(wid * tpw, tpw)], colb_v)

    sems = (sem0, sem1)
    bufs = (rows0_v, rows1_v)

    def gather(g, slot):
        idx = row_v.at[pl.ds(g * _CHUNK, _CHUNK)]
        return pltpu.async_copy(p2_hbm.at[idx], bufs[slot], sems[slot])

    def gather_wait(g, slot):
        idx = row_v.at[pl.ds(g * _CHUNK, _CHUNK)]
        pltpu.make_async_copy(p2_hbm.at[idx], bufs[slot], sems[slot]).wait()

    gather(0, 0)
    gather(1, 1)

    lane = lax.iota(jnp.int32, 16)

    def row_body(r, _):
        accT = (jnp.zeros((16,), jnp.float32),) * 16
        for c in range(_NCH):
            g = r * _NCH + c
            slot = c % 2
            gather_wait(g, slot)

            @pl.when(g + 2 < _NG)
            def _():
                gather(g + 2, slot)

            buf = bufs[slot]

            def group_body(gi, a):
                off = g * _CHUNK + gi * 16
                colb = colb_v[pl.ds(off, 16)]
                rowi = lane + gi * 16
                return tuple(
                    a[l] + plsc.load_gather(buf, [rowi, colb + l])
                    for l in range(16)
                )

            accT = lax.fori_loop(0, _CHUNK // 16, group_body, accT)

        for l in range(16):
            plsc.store_scatter(tr_v, [lane, jnp.full((16,), l, jnp.int32)],
                               accT[l])
        out = tr_v[0, :]
        for k in range(1, 16):
            out = out + tr_v[k, :]
        out_v[pl.ds(r * _HIDDEN, _HIDDEN)] = out
        return 0

    lax.fori_loop(0, _RPW, row_body, 0)
    opw = _RPW * _HIDDEN
    pltpu.sync_copy(out_v, out_hbm.at[pl.ds(wid * opw, opw)])


_bag = functools.partial(
    pl.kernel,
    out_type=jax.ShapeDtypeStruct((_BATCH * _HIDDEN,), jnp.float32),
    mesh=plsc.VectorSubcoreMesh(core_axis_name="c", subcore_axis_name="s"),
    scratch_types=[
        pltpu.VMEM((_RPW * _SEQ,), jnp.int32),
        pltpu.VMEM((_RPW * _SEQ,), jnp.int32),
        pltpu.VMEM((_CHUNK, 128), jnp.float32),
        pltpu.VMEM((_CHUNK, 128), jnp.float32),
        pltpu.VMEM((16, 16), jnp.float32),
        pltpu.VMEM((_RPW * _HIDDEN,), jnp.float32),
        pltpu.SemaphoreType.DMA,
        pltpu.SemaphoreType.DMA,
    ],
    compiler_params=pltpu.CompilerParams(needs_layout_passes=False),
)(_bag_body)


def _head_body(h_ref, b1_ref, w2_ref, b2_ref, o_ref):
    h = jnp.maximum(h_ref[:] + b1_ref[:], 0.0)
    z = jnp.dot(h, w2_ref[:], preferred_element_type=jnp.float32) + b2_ref[:]
    o_ref[:] = 1.0 / (1.0 + jnp.exp(-z))


def kernel(tokens, emb_table, W1, b1, W2, b2):
    w1s = W1 * jnp.float32(1.0 / _SEQ)
    w1bd = jnp.zeros((_NBAND * _EMBED, 128), jnp.float32)
    for j in range(_NBAND):
        w1bd = w1bd.at[j * _EMBED:(j + 1) * _EMBED,
                       j * _HIDDEN:(j + 1) * _HIDDEN].set(w1s)
    w1bd = w1bd.astype(jnp.bfloat16)

    p2, grow, gcolb = _proj(*([emb_table] * _NBAND), w1bd,
                            tokens.reshape(-1, 128))
    h_sum = _bag(grow.reshape(-1), gcolb.reshape(-1), p2)
    out = pl.pallas_call(
        _head_body,
        out_shape=jax.ShapeDtypeStruct((_BATCH, 1), jnp.float32),
    )(h_sum.reshape(_BATCH, _HIDDEN), b1.reshape(1, _HIDDEN),
      W2, b2.reshape(1, 1))
    return out


# speedup vs baseline: 4.2792x; 1.0089x over previous
import functools

import jax
import jax.numpy as jnp
from jax import lax
from jax.experimental import pallas as pl
from jax.experimental.pallas import tpu as pltpu
from jax.experimental.pallas import tpu_sc as plsc

_VOCAB = 1000000
_EMBED = 300
_BATCH = 1024
_SEQ = 512
_HIDDEN = 16

_NC, _NS = 2, 16
_NW = _NC * _NS
_RPW = _BATCH // _NW
_CHUNK = 128
_NCH = _SEQ // _CHUNK
_NG = _RPW * _NCH
_BAND = 1000
_NBAND = 128 // _HIDDEN
_PBLK = _BAND * _NBAND
_P2ROWS = _VOCAB // _NBAND


_TROWS = _BATCH * _SEQ // 128


def _proj_body(*refs):
    xs, w_ref, t_ref = refs[:_NBAND], refs[_NBAND], refs[_NBAND + 1]
    o_ref, row_ref, colb_ref = refs[_NBAND + 2:]
    acc = jnp.zeros((_BAND, 128), jnp.float32)
    for j in range(_NBAND):
        wj = w_ref[pl.ds(j * _EMBED, _EMBED), :]
        acc = acc + jnp.dot(xs[j][:].astype(jnp.bfloat16), wj,
                            preferred_element_type=jnp.float32)
    o_ref[:] = acc

    @pl.when(pl.program_id(0) == 0)
    def _():
        t = t_ref[:]
        blk = t // _PBLK
        band = (t // _BAND) % _NBAND
        r = t % _BAND
        row_ref[:] = blk * _BAND + r
        colb_ref[:] = band * _HIDDEN


_proj = pl.pallas_call(
    _proj_body,
    grid=(_VOCAB // _PBLK,),
    in_specs=[
        pl.BlockSpec((_BAND, _EMBED), lambda i, j=j: (_NBAND * i + j, 0))
        for j in range(_NBAND)
    ] + [
        pl.BlockSpec((_NBAND * _EMBED, 128), lambda i: (0, 0)),
        pl.BlockSpec((_TROWS, 128), lambda i: (0, 0)),
    ],
    out_specs=[
        pl.BlockSpec((_BAND, 128), lambda i: (i, 0)),
        pl.BlockSpec((_TROWS, 128), lambda i: (0, 0)),
        pl.BlockSpec((_TROWS, 128), lambda i: (0, 0)),
    ],
    out_shape=[
        jax.ShapeDtypeStruct((_P2ROWS, 128), jnp.float32),
        jax.ShapeDtypeStruct((_TROWS, 128), jnp.int32),
        jax.ShapeDtypeStruct((_TROWS, 128), jnp.int32),
    ],
)


_NBUF = 4


def _bag_body(row_hbm, colb_hbm, p2_hbm, out_hbm,
              row_v, colb_v, rows0_v, rows1_v, rows2_v, rows3_v, tr_v, out_v,
              sem0, sem1, sem2, sem3):
    wid = lax.axis_index("s") * _NC + lax.axis_index("c")
    tpw = _RPW * _SEQ
    pltpu.sync_copy(row_hbm.at[pl.ds(wid * tpw, tpw)], row_v)
    pltpu.sync_copy(colb_hbm.at[pl.ds(wid * tpw, tpw)], colb_v)

    sems = (sem0, sem1, sem2, sem3)
    bufs = (rows0_v, rows1_v, rows2_v, rows3_v)

    def gather(g, slot):
        idx = row_v.at[pl.ds(g * _CHUNK, _CHUNK)]
        return pltpu.async_copy(p2_hbm.at[idx], bufs[slot], sems[slot])

    def gather_wait(g, slot):
        idx = row_v.at[pl.ds(g * _CHUNK, _CHUNK)]
        pltpu.make_async_copy(p2_hbm.at[idx], bufs[slot], sems[slot]).wait()

    for s in range(_NBUF):
        gather(s, s)

    lane = lax.iota(jnp.int32, 16)

    def row_body(r, _):
        accT = (jnp.zeros((16,), jnp.float32),) * 16
        for c in range(_NCH):
            g = r * _NCH + c
            slot = c % _NBUF
            gather_wait(g, slot)

            @pl.when(g + _NBUF < _NG)
            def _():
                gather(g + _NBUF, slot)

            buf = bufs[slot]

            def group_body(gi, a):
                off = g * _CHUNK + gi * 16
                colb = colb_v[pl.ds(off, 16)]
                rowi = lane + gi * 16
                return tuple(
                    a[l] + plsc.load_gather(buf, [rowi, colb + l])
                    for l in range(16)
                )

            accT = lax.fori_loop(0, _CHUNK // 16, group_body, accT)

        for l in range(16):
            plsc.store_scatter(tr_v, [lane, jnp.full((16,), l, jnp.int32)],
                               accT[l])
        out = tr_v[0, :]
        for k in range(1, 16):
            out = out + tr_v[k, :]
        out_v[pl.ds(r * _HIDDEN, _HIDDEN)] = out
        return 0

    lax.fori_loop(0, _RPW, row_body, 0)
    opw = _RPW * _HIDDEN
    pltpu.sync_copy(out_v, out_hbm.at[pl.ds(wid * opw, opw)])


_bag = functools.partial(
    pl.kernel,
    out_type=jax.ShapeDtypeStruct((_BATCH * _HIDDEN,), jnp.float32),
    mesh=plsc.VectorSubcoreMesh(core_axis_name="c", subcore_axis_name="s"),
    scratch_types=[
        pltpu.VMEM((_RPW * _SEQ,), jnp.int32),
        pltpu.VMEM((_RPW * _SEQ,), jnp.int32),
        pltpu.VMEM((_CHUNK, 128), jnp.float32),
        pltpu.VMEM((_CHUNK, 128), jnp.float32),
        pltpu.VMEM((_CHUNK, 128), jnp.float32),
        pltpu.VMEM((_CHUNK, 128), jnp.float32),
        pltpu.VMEM((16, 16), jnp.float32),
        pltpu.VMEM((_RPW * _HIDDEN,), jnp.float32),
        pltpu.SemaphoreType.DMA,
        pltpu.SemaphoreType.DMA,
        pltpu.SemaphoreType.DMA,
        pltpu.SemaphoreType.DMA,
    ],
    compiler_params=pltpu.CompilerParams(needs_layout_passes=False),
)(_bag_body)


def _head_body(h_ref, b1_ref, w2_ref, b2_ref, o_ref):
    h = jnp.maximum(h_ref[:] + b1_ref[:], 0.0)
    z = jnp.dot(h, w2_ref[:], preferred_element_type=jnp.float32) + b2_ref[:]
    o_ref[:] = 1.0 / (1.0 + jnp.exp(-z))


def kernel(tokens, emb_table, W1, b1, W2, b2):
    w1s = W1 * jnp.float32(1.0 / _SEQ)
    w1bd = jnp.zeros((_NBAND * _EMBED, 128), jnp.float32)
    for j in range(_NBAND):
        w1bd = w1bd.at[j * _EMBED:(j + 1) * _EMBED,
                       j * _HIDDEN:(j + 1) * _HIDDEN].set(w1s)
    w1bd = w1bd.astype(jnp.bfloat16)

    p2, grow, gcolb = _proj(*([emb_table] * _NBAND), w1bd,
                            tokens.reshape(-1, 128))
    h_sum = _bag(grow.reshape(-1), gcolb.reshape(-1), p2)
    out = pl.pallas_call(
        _head_body,
        out_shape=jax.ShapeDtypeStruct((_BATCH, 1), jnp.float32),
    )(h_sum.reshape(_BATCH, _HIDDEN), b1.reshape(1, _HIDDEN),
      W2, b2.reshape(1, 1))
    return out


# speedup vs baseline: 4.2880x; 1.0020x over previous
import functools

import jax
import jax.numpy as jnp
from jax import lax
from jax.experimental import pallas as pl
from jax.experimental.pallas import tpu as pltpu
from jax.experimental.pallas import tpu_sc as plsc

_VOCAB = 1000000
_EMBED = 300
_BATCH = 1024
_SEQ = 512
_HIDDEN = 16

_NC, _NS = 2, 16
_NW = _NC * _NS
_RPW = _BATCH // _NW
_CHUNK = 128
_NCH = _SEQ // _CHUNK
_NG = _RPW * _NCH
_BAND = 1000
_NBAND = 128 // _HIDDEN
_PBLK = _BAND * _NBAND
_P2ROWS = _VOCAB // _NBAND


_TROWS = _BATCH * _SEQ // 128


def _proj_body(x_ref, w_ref, t_ref, o_ref, row_ref, colb_ref):
    acc = jnp.zeros((_BAND, 128), jnp.float32)
    for j in range(_NBAND):
        xj = x_ref[pl.ds(j * _BAND, _BAND), :]
        wj = w_ref[pl.ds(j * _EMBED, _EMBED), :]
        acc = acc + jnp.dot(xj.astype(jnp.bfloat16), wj,
                            preferred_element_type=jnp.float32)
    o_ref[:] = acc

    @pl.when(pl.program_id(0) == 0)
    def _():
        t = t_ref[:]
        blk = t // _PBLK
        band = (t // _BAND) % _NBAND
        r = t % _BAND
        row_ref[:] = blk * _BAND + r
        colb_ref[:] = band * _HIDDEN


_proj = pl.pallas_call(
    _proj_body,
    grid=(_VOCAB // _PBLK,),
    in_specs=[
        pl.BlockSpec((_PBLK, _EMBED), lambda i: (i, 0)),
        pl.BlockSpec((_NBAND * _EMBED, 128), lambda i: (0, 0)),
        pl.BlockSpec((_TROWS, 128), lambda i: (0, 0)),
    ],
    out_specs=[
        pl.BlockSpec((_BAND, 128), lambda i: (i, 0)),
        pl.BlockSpec((_TROWS, 128), lambda i: (0, 0)),
        pl.BlockSpec((_TROWS, 128), lambda i: (0, 0)),
    ],
    out_shape=[
        jax.ShapeDtypeStruct((_P2ROWS, 128), jnp.float32),
        jax.ShapeDtypeStruct((_TROWS, 128), jnp.int32),
        jax.ShapeDtypeStruct((_TROWS, 128), jnp.int32),
    ],
)


_NBUF = 4


def _bag_body(row_hbm, colb_hbm, p2_hbm, out_hbm,
              row_v, colb_v, rows0_v, rows1_v, rows2_v, rows3_v, tr_v, out_v,
              sem0, sem1, sem2, sem3):
    wid = lax.axis_index("s") * _NC + lax.axis_index("c")
    tpw = _RPW * _SEQ
    pltpu.sync_copy(row_hbm.at[pl.ds(wid * tpw, tpw)], row_v)
    pltpu.sync_copy(colb_hbm.at[pl.ds(wid * tpw, tpw)], colb_v)

    sems = (sem0, sem1, sem2, sem3)
    bufs = (rows0_v, rows1_v, rows2_v, rows3_v)

    def gather(g, slot):
        idx = row_v.at[pl.ds(g * _CHUNK, _CHUNK)]
        return pltpu.async_copy(p2_hbm.at[idx], bufs[slot], sems[slot])

    def gather_wait(g, slot):
        idx = row_v.at[pl.ds(g * _CHUNK, _CHUNK)]
        pltpu.make_async_copy(p2_hbm.at[idx], bufs[slot], sems[slot]).wait()

    for s in range(_NBUF):
        gather(s, s)

    lane = lax.iota(jnp.int32, 16)

    def row_body(r, _):
        accT = (jnp.zeros((16,), jnp.float32),) * 16
        for c in range(_NCH):
            g = r * _NCH + c
            slot = c % _NBUF
            gather_wait(g, slot)

            @pl.when(g + _NBUF < _NG)
            def _():
                gather(g + _NBUF, slot)

            buf = bufs[slot]

            def group_body(gi, a):
                off = g * _CHUNK + gi * 16
                colb = colb_v[pl.ds(off, 16)]
                rowi = lane + gi * 16
                return tuple(
                    a[l] + plsc.load_gather(buf, [rowi, colb + l])
                    for l in range(16)
                )

            accT = lax.fori_loop(0, _CHUNK // 16, group_body, accT)

        for l in range(16):
            plsc.store_scatter(tr_v, [lane, jnp.full((16,), l, jnp.int32)],
                               accT[l])
        out = tr_v[0, :]
        for k in range(1, 16):
            out = out + tr_v[k, :]
        out_v[pl.ds(r * _HIDDEN, _HIDDEN)] = out
        return 0

    lax.fori_loop(0, _RPW, row_body, 0)
    opw = _RPW * _HIDDEN
    pltpu.sync_copy(out_v, out_hbm.at[pl.ds(wid * opw, opw)])


_bag = functools.partial(
    pl.kernel,
    out_type=jax.ShapeDtypeStruct((_BATCH * _HIDDEN,), jnp.float32),
    mesh=plsc.VectorSubcoreMesh(core_axis_name="c", subcore_axis_name="s"),
    scratch_types=[
        pltpu.VMEM((_RPW * _SEQ,), jnp.int32),
        pltpu.VMEM((_RPW * _SEQ,), jnp.int32),
        pltpu.VMEM((_CHUNK, 128), jnp.float32),
        pltpu.VMEM((_CHUNK, 128), jnp.float32),
        pltpu.VMEM((_CHUNK, 128), jnp.float32),
        pltpu.VMEM((_CHUNK, 128), jnp.float32),
        pltpu.VMEM((16, 16), jnp.float32),
        pltpu.VMEM((_RPW * _HIDDEN,), jnp.float32),
        pltpu.SemaphoreType.DMA,
        pltpu.SemaphoreType.DMA,
        pltpu.SemaphoreType.DMA,
        pltpu.SemaphoreType.DMA,
    ],
    compiler_params=pltpu.CompilerParams(needs_layout_passes=False),
)(_bag_body)


def _head_body(h_ref, b1_ref, w2_ref, b2_ref, o_ref):
    h = jnp.maximum(h_ref[:] + b1_ref[:], 0.0)
    z = jnp.dot(h, w2_ref[:], preferred_element_type=jnp.float32) + b2_ref[:]
    o_ref[:] = 1.0 / (1.0 + jnp.exp(-z))


def kernel(tokens, emb_table, W1, b1, W2, b2):
    w1s = W1 * jnp.float32(1.0 / _SEQ)
    w1bd = jnp.zeros((_NBAND * _EMBED, 128), jnp.float32)
    for j in range(_NBAND):
        w1bd = w1bd.at[j * _EMBED:(j + 1) * _EMBED,
                       j * _HIDDEN:(j + 1) * _HIDDEN].set(w1s)
    w1bd = w1bd.astype(jnp.bfloat16)

    p2, grow, gcolb = _proj(emb_table, w1bd, tokens.reshape(-1, 128))
    h_sum = _bag(grow.reshape(-1), gcolb.reshape(-1), p2)
    out = pl.pallas_call(
        _head_body,
        out_shape=jax.ShapeDtypeStruct((_BATCH, 1), jnp.float32),
    )(h_sum.reshape(_BATCH, _HIDDEN), b1.reshape(1, _HIDDEN),
      W2, b2.reshape(1, 1))
    return out
